# baseline (speedup 1.0000x reference)
import functools

import jax
import jax.numpy as jnp
from jax import lax
from jax.experimental import pallas as pl
from jax.experimental.pallas import tpu as pltpu
from jax.experimental.pallas import tpu_sc as plsc

D_MODEL = 1024
NUM_ROWS_TOTAL = 4 * 8192
NUM_CORES = 2
NUM_SUBCORES = 16
NUM_WORKERS = NUM_CORES * NUM_SUBCORES
B_PER_W = NUM_ROWS_TOTAL // NUM_WORKERS
CHUNK = 32
NBUF = 2
NUM_CHUNKS = B_PER_W // CHUNK
GROUPS = NUM_CHUNKS // NBUF

_mesh = plsc.VectorSubcoreMesh(core_axis_name="c", subcore_axis_name="s")


@functools.partial(
    pl.kernel,
    out_type=jax.ShapeDtypeStruct((NUM_ROWS_TOTAL, D_MODEL), jnp.float32),
    mesh=_mesh,
    scratch_types=[
        pltpu.VMEM((B_PER_W,), jnp.int32),
        pltpu.VMEM((NBUF, CHUNK, D_MODEL), jnp.float32),
        pltpu.SemaphoreType.DMA,
        pltpu.SemaphoreType.DMA,
        pltpu.SemaphoreType.DMA,
        pltpu.SemaphoreType.DMA,
    ],
)
def _embed_sc(table_hbm, idx_hbm, out_hbm, idx_v, rows_v, g0, g1, s0, s1):
    wid = lax.axis_index("s") * NUM_CORES + lax.axis_index("c")
    base = wid * B_PER_W
    pltpu.sync_copy(idx_hbm.at[pl.ds(base, B_PER_W)], idx_v)
    gsems = (g0, g1)
    ssems = (s0, s1)

    def start_gather(c, b):
        pltpu.async_copy(
            table_hbm.at[idx_v.at[pl.ds(c * CHUNK, CHUNK)]],
            rows_v.at[b],
            gsems[b],
        )

    def wait_gather(b):
        pltpu.make_async_copy(
            table_hbm.at[pl.ds(0, CHUNK)], rows_v.at[b], gsems[b]
        ).wait()

    def start_scatter(c, b):
        pltpu.async_copy(
            rows_v.at[b], out_hbm.at[pl.ds(base + c * CHUNK, CHUNK)], ssems[b]
        )

    def wait_scatter(b):
        pltpu.make_async_copy(
            rows_v.at[b], out_hbm.at[pl.ds(base, CHUNK)], ssems[b]
        ).wait()

    for b in range(NBUF):
        start_gather(b, b)

    def group_body(g, carry):
        for b in range(NBUF):
            c = g * NBUF + b
            wait_gather(b)
            start_scatter(c, b)
        for b in range(NBUF):
            c = g * NBUF + b
            wait_scatter(b)
            start_gather(c + NBUF, b)
        return carry

    lax.fori_loop(0, GROUPS - 1, group_body, 0)

    g_last = GROUPS - 1
    for b in range(NBUF):
        c = g_last * NBUF + b
        wait_gather(b)
        start_scatter(c, b)
    for b in range(NBUF):
        wait_scatter(b)


def kernel(x, table):
    flat_idx = x.reshape(-1).astype(jnp.int32)
    out = _embed_sc(table, flat_idx)
    return out.reshape(x.shape + (table.shape[1],))


# speedup vs baseline: 1.1246x; 1.1246x over previous
import functools

import jax
import jax.numpy as jnp
from jax import lax
from jax.experimental import pallas as pl
from jax.experimental.pallas import tpu as pltpu
from jax.experimental.pallas import tpu_sc as plsc

D_MODEL = 1024
NUM_COLORS = 16
NUM_ROWS_TOTAL = 4 * 8192
NUM_CORES = 2
NUM_SUBCORES = 16
NUM_WORKERS = NUM_CORES * NUM_SUBCORES
B_PER_W = NUM_ROWS_TOTAL // NUM_WORKERS
CHUNK = 32
NBUF = 2
NUM_CHUNKS = B_PER_W // CHUNK
LANES = 16
VECS_PER_ROW = D_MODEL // LANES

_mesh = plsc.VectorSubcoreMesh(core_axis_name="c", subcore_axis_name="s")


@functools.partial(
    pl.kernel,
    out_type=jax.ShapeDtypeStruct((NUM_ROWS_TOTAL, D_MODEL), jnp.float32),
    mesh=_mesh,
    scratch_types=[
        pltpu.VMEM((NUM_COLORS, D_MODEL), jnp.float32),
        pltpu.VMEM((B_PER_W,), jnp.int32),
        pltpu.VMEM((NBUF * CHUNK, D_MODEL), jnp.float32),
        pltpu.SemaphoreType.DMA,
        pltpu.SemaphoreType.DMA,
    ],
)
def _embed_sc(table_hbm, idx_hbm, out_hbm, table_v, idx_v, rows_v, s0, s1):
    wid = lax.axis_index("s") * NUM_CORES + lax.axis_index("c")
    base = wid * B_PER_W
    pltpu.sync_copy(table_hbm, table_v)
    pltpu.sync_copy(idx_hbm.at[pl.ds(base, B_PER_W)], idx_v)

    def fill_chunk(c):
        row0 = (c % NBUF) * CHUNK

        def grp_body(g, carry):
            vec = idx_v[pl.ds(c * CHUNK + g * LANES, LANES)]
            for k in range(LANES):
                v = vec[k]
                dst = row0 + g * LANES + k
                for j in range(VECS_PER_ROW):
                    sl = pl.ds(j * LANES, LANES)
                    rows_v[dst, sl] = table_v[v, sl]
            return carry

        lax.fori_loop(0, CHUNK // LANES, grp_body, 0)

    def start_scatter(c, b, sem):
        pltpu.async_copy(
            rows_v.at[pl.ds(b * CHUNK, CHUNK)],
            out_hbm.at[pl.ds(base + c * CHUNK, CHUNK)],
            sem,
        )

    def wait_scatter(b, sem):
        pltpu.make_async_copy(
            rows_v.at[pl.ds(b * CHUNK, CHUNK)],
            out_hbm.at[pl.ds(0, CHUNK)],
            sem,
        ).wait()

    def chunk_body(c, carry):
        parity = c % NBUF

        @pl.when(jnp.logical_and(c >= NBUF, parity == 0))
        def _():
            wait_scatter(0, s0)

        @pl.when(jnp.logical_and(c >= NBUF, parity == 1))
        def _():
            wait_scatter(1, s1)

        fill_chunk(c)

        @pl.when(parity == 0)
        def _():
            start_scatter(c, 0, s0)

        @pl.when(parity == 1)
        def _():
            start_scatter(c, 1, s1)

        return carry

    lax.fori_loop(0, NUM_CHUNKS, chunk_body, 0)
    wait_scatter(0, s0)
    wait_scatter(1, s1)


def kernel(x, table):
    flat_idx = x.reshape(-1).astype(jnp.int32)
    out = _embed_sc(table, flat_idx)
    return out.reshape(x.shape + (table.shape[1],))


# speedup vs baseline: 4.4420x; 3.9498x over previous
import functools

import jax
import jax.numpy as jnp
from jax import lax
from jax.experimental import pallas as pl
from jax.experimental.pallas import tpu as pltpu
from jax.experimental.pallas import tpu_sc as plsc

D_MODEL = 1024
NUM_COLORS = 16
NUM_ROWS_TOTAL = 4 * 8192
NUM_CORES = 2
NUM_SUBCORES = 16
NUM_WORKERS = NUM_CORES * NUM_SUBCORES
B_PER_W = NUM_ROWS_TOTAL // NUM_WORKERS
CHUNK = 32
NBUF = 2
NUM_CHUNKS = B_PER_W // CHUNK
LANES = 16

_mesh = plsc.VectorSubcoreMesh(core_axis_name="c", subcore_axis_name="s")


@functools.partial(
    pl.kernel,
    out_type=jax.ShapeDtypeStruct((NUM_ROWS_TOTAL, D_MODEL), jnp.float32),
    mesh=_mesh,
    scratch_types=[
        pltpu.VMEM_SHARED((NUM_COLORS, D_MODEL), jnp.float32),
        pltpu.VMEM((B_PER_W,), jnp.int32),
        pltpu.VMEM((NBUF * CHUNK, D_MODEL), jnp.float32),
        pltpu.SemaphoreType.DMA,
        pltpu.SemaphoreType.DMA,
        pltpu.SemaphoreType.DMA,
    ],
)
def _embed_sc(table_hbm, idx_hbm, out_hbm, table_v, idx_v, rows_v, fsem, s0, s1):
    sid = lax.axis_index("s")
    wid = sid * NUM_CORES + lax.axis_index("c")
    base = wid * B_PER_W

    @pl.when(sid == 0)
    def _():
        pltpu.sync_copy(table_hbm, table_v)

    pltpu.sync_copy(idx_hbm.at[pl.ds(base, B_PER_W)], idx_v)
    plsc.subcore_barrier()

    def fill_chunk(c):
        row0 = (c % NBUF) * CHUNK

        def grp_body(g, carry):
            vec = idx_v[pl.ds(c * CHUNK + g * LANES, LANES)]
            for k in range(LANES):
                v = vec[k]
                dst = row0 + g * LANES + k
                pltpu.async_copy(
                    table_v.at[pl.ds(v, 1)], rows_v.at[pl.ds(dst, 1)], fsem
                )
            return carry

        lax.fori_loop(0, CHUNK // LANES, grp_body, 0)
        pltpu.make_async_copy(
            out_hbm.at[pl.ds(0, CHUNK)], rows_v.at[pl.ds(0, CHUNK)], fsem
        ).wait()

    def start_scatter(c, b, sem):
        pltpu.async_copy(
            rows_v.at[pl.ds(b * CHUNK, CHUNK)],
            out_hbm.at[pl.ds(base + c * CHUNK, CHUNK)],
            sem,
        )

    def wait_scatter(b, sem):
        pltpu.make_async_copy(
            rows_v.at[pl.ds(b * CHUNK, CHUNK)],
            out_hbm.at[pl.ds(0, CHUNK)],
            sem,
        ).wait()

    def chunk_body(c, carry):
        parity = c % NBUF

        @pl.when(jnp.logical_and(c >= NBUF, parity == 0))
        def _():
            wait_scatter(0, s0)

        @pl.when(jnp.logical_and(c >= NBUF, parity == 1))
        def _():
            wait_scatter(1, s1)

        fill_chunk(c)

        @pl.when(parity == 0)
        def _():
            start_scatter(c, 0, s0)

        @pl.when(parity == 1)
        def _():
            start_scatter(c, 1, s1)

        return carry

    lax.fori_loop(0, NUM_CHUNKS, chunk_body, 0)
    wait_scatter(0, s0)
    wait_scatter(1, s1)


def kernel(x, table):
    flat_idx = x.reshape(-1).astype(jnp.int32)
    out = _embed_sc(table, flat_idx)
    return out.reshape(x.shape + (table.shape[1],))
